# Initial kernel scaffold; baseline (speedup 1.0000x reference)
import jax
import jax.numpy as jnp
from jax.experimental import pallas as pl


def kernel(tokens, token_emb, pos_emb, Wr, Wq, Wk, Wv, Wo, g1, g2, W1, b1, W2, b2):
    raise NotImplementedError("write your pallas kernel here")


# speedup vs baseline: 1.5663x; 1.5663x over previous
import functools

import jax
import jax.numpy as jnp
from jax import lax
from jax.experimental import pallas as pl
from jax.experimental.pallas import tpu as pltpu
from jax.experimental.pallas import tpu_sc as plsc

T = 2048
D = 1024
H = 16
DH = 64
K = 512
L = 4
V = 32000
EPS = 1.1920928955078125e-07
INT_MIN = -2147483648
NEG = -1e30


_NW = 32
_BPW = T // _NW


def _sc_embed_gather(table, idx):
    mesh = plsc.VectorSubcoreMesh(core_axis_name="c", subcore_axis_name="s")

    @functools.partial(
        pl.kernel,
        mesh=mesh,
        out_type=jax.ShapeDtypeStruct((T, D), jnp.float32),
        scratch_types=[
            pltpu.VMEM((_BPW,), jnp.int32),
            pltpu.VMEM((_BPW, D), jnp.float32),
            pltpu.SemaphoreType.DMA,
        ],
    )
    def k(table_hbm, idx_hbm, out_hbm, idx_v, rows_v, sem):
        wid = lax.axis_index("s") * 2 + lax.axis_index("c")
        base = wid * _BPW
        pltpu.sync_copy(idx_hbm.at[pl.ds(base, _BPW)], idx_v)
        pltpu.async_copy(table_hbm.at[idx_v], rows_v, sem).wait()
        pltpu.sync_copy(rows_v, out_hbm.at[pl.ds(base, _BPW)])

    return k(table, idx)


def _add_kernel(a_ref, b_ref, o_ref):
    o_ref[...] = a_ref[...] + b_ref[...]


def _add(a, b):
    return pl.pallas_call(
        _add_kernel,
        out_shape=jax.ShapeDtypeStruct(a.shape, a.dtype),
    )(a, b)


def _cumsum_lanes(a, n):
    s = 1
    while s < n:
        pad = a[..., :s] * 0.0
        a = a + jnp.concatenate([pad, a[..., : n - s]], axis=-1)
        s *= 2
    return a


def _norm_route_kernel(x_ref, g1_ref, wr_ref, xn_ref, sel_ref, csum_ref):
    x = x_ref[...]
    ms = jnp.mean(x * x, axis=1, keepdims=True)
    xn = x * lax.rsqrt(ms + EPS) * g1_ref[...]
    xn_ref[...] = xn
    rs = lax.dot_general(
        wr_ref[...], xn, (((1,), (1,)), ((), ())),
        preferred_element_type=jnp.float32)
    bits = lax.bitcast_convert_type(rs, jnp.int32)
    key = jnp.where(bits >= 0, bits, INT_MIN - bits)
    cnt0 = jnp.sum((key >= 0).astype(jnp.int32), axis=1, keepdims=True)
    prefix0 = jnp.where(cnt0 >= K, 0, INT_MIN)

    def body(i, prefix):
        cand = prefix | jnp.left_shift(1, 30 - i)
        cnt = jnp.sum((key >= cand).astype(jnp.int32), axis=1, keepdims=True)
        return jnp.where(cnt >= K, cand, prefix)

    thr = lax.fori_loop(0, 31, body, prefix0)
    gt = key > thr
    c1 = jnp.sum(gt.astype(jnp.int32), axis=1, keepdims=True)
    eq = key == thr
    eqf = eq.astype(jnp.float32)
    rank = _cumsum_lanes(eqf, T)
    need = (K - c1).astype(jnp.float32)
    sel = gt | (eq & (rank <= need))
    self_f = sel.astype(jnp.float32)
    sel_ref[...] = self_f.reshape(H, 1, T)
    csum_ref[...] = _cumsum_lanes(self_f, T).reshape(H, 1, T)


def _norm_route(x, g1, wr):
    return pl.pallas_call(
        _norm_route_kernel,
        out_shape=(
            jax.ShapeDtypeStruct((T, D), jnp.float32),
            jax.ShapeDtypeStruct((H, 1, T), jnp.float32),
            jax.ShapeDtypeStruct((H, 1, T), jnp.float32),
        ),
    )(x, g1, wr)


def _qkv_kernel(xn_ref, wq_ref, wk_ref, wv_ref, q_ref, k_ref, v_ref):
    xn = xn_ref[...]
    dn = (((1,), (1,)), ((), ()))
    q_ref[...] = lax.dot_general(xn, wq_ref[...], dn,
                                 preferred_element_type=jnp.float32)
    k_ref[...] = lax.dot_general(xn, wk_ref[...], dn,
                                 preferred_element_type=jnp.float32)
    v_ref[...] = lax.dot_general(xn, wv_ref[...], dn,
                                 preferred_element_type=jnp.float32)


def _qkv(xn, wq, wk, wv):
    return pl.pallas_call(
        _qkv_kernel,
        out_shape=(
            jax.ShapeDtypeStruct((T, D), jnp.float32),
            jax.ShapeDtypeStruct((T, D), jnp.float32),
            jax.ShapeDtypeStruct((T, D), jnp.float32),
        ),
    )(xn, wq, wk, wv)


def _head_attn_kernel(q_ref, k_ref, v_ref, sel_ref, csum_ref, o_ref):
    q = q_ref[...].reshape(T, DH)
    k = k_ref[...].reshape(T, DH)
    v = v_ref[...].reshape(T, DH)
    sel = sel_ref[...].reshape(1, T)
    csum = csum_ref[...].reshape(1, T)
    i0 = lax.broadcasted_iota(jnp.int32, (K, T), 0).astype(jnp.float32) + 1.0
    g = jnp.where((i0 == csum) & (sel > 0.5), 1.0, 0.0).astype(jnp.float32)
    dn_r = (((1,), (0,)), ((), ()))
    qh = lax.dot_general(g, q, dn_r, preferred_element_type=jnp.float32)
    kh = lax.dot_general(g, k, dn_r, preferred_element_type=jnp.float32)
    vh = lax.dot_general(g, v, dn_r, preferred_element_type=jnp.float32)
    ar_col = lax.broadcasted_iota(jnp.int32, (T, 1), 0).astype(jnp.float32)
    ar_row = lax.broadcasted_iota(jnp.int32, (1, T), 1).astype(jnp.float32)
    pos_col = lax.dot_general(g, ar_col, dn_r,
                              preferred_element_type=jnp.float32)
    pos_row = lax.dot_general(ar_row, g, (((1,), (1,)), ((), ())),
                              preferred_element_type=jnp.float32)
    sc = lax.dot_general(qh, kh, (((1,), (1,)), ((), ())),
                         preferred_element_type=jnp.float32)
    sc = sc * (DH ** -0.5)
    sc = jnp.where(pos_col >= pos_row, sc, NEG)
    m = jnp.max(sc, axis=1, keepdims=True)
    p = jnp.exp(sc - m)
    at = p / jnp.sum(p, axis=1, keepdims=True)
    oh = lax.dot_general(at, vh, (((1,), (0,)), ((), ())),
                         preferred_element_type=jnp.float32)
    out = lax.dot_general(g, oh, (((0,), (0,)), ((), ())),
                          preferred_element_type=jnp.float32)
    o_ref[...] = out.reshape(1, T, DH)


def _head_attn(q3, k3, v3, sel3, csum3):
    blk_h = pl.BlockSpec((1, T, DH), lambda h: (h, 0, 0))
    blk_s = pl.BlockSpec((1, 1, T), lambda h: (h, 0, 0))
    return pl.pallas_call(
        _head_attn_kernel,
        grid=(H,),
        in_specs=[blk_h, blk_h, blk_h, blk_s, blk_s],
        out_specs=blk_h,
        out_shape=jax.ShapeDtypeStruct((H, T, DH), jnp.float32),
    )(q3, k3, v3, sel3, csum3)


def _mlp_kernel(x_ref, a_ref, wo_ref, g2_ref, w1_ref, b1_ref, w2_ref,
                b2_ref, y_ref):
    dn = (((1,), (1,)), ((), ()))
    x1 = x_ref[...] + lax.dot_general(a_ref[...], wo_ref[...], dn,
                                      preferred_element_type=jnp.float32)
    ms = jnp.mean(x1 * x1, axis=1, keepdims=True)
    h = x1 * lax.rsqrt(ms + EPS) * g2_ref[...]
    h1 = lax.dot_general(h, w1_ref[...], dn,
                         preferred_element_type=jnp.float32) + b1_ref[...]
    h1 = h1 / (1.0 + jnp.exp(-h1))
    y_ref[...] = x1 + lax.dot_general(h1, w2_ref[...], dn,
                                      preferred_element_type=jnp.float32) \
        + b2_ref[...]


def _oproj_mlp(x, attn, wo, g2, w1, b1, w2, b2):
    rb = 512
    row = pl.BlockSpec((rb, D), lambda i: (i, 0))
    full = lambda shape: pl.BlockSpec(shape, lambda i: (0,) * len(shape))
    return pl.pallas_call(
        _mlp_kernel,
        grid=(T // rb,),
        in_specs=[row, row, full((D, D)), full((1, D)), full((4 * D, D)),
                  full((1, 4 * D)), full((D, 4 * D)), full((1, D))],
        out_specs=row,
        out_shape=jax.ShapeDtypeStruct((T, D), jnp.float32),
    )(x, attn, wo, g2, w1, b1, w2, b2)


def _lm_head_kernel(x_ref, e_ref, o_ref):
    o_ref[...] = lax.dot_general(x_ref[...], e_ref[...],
                                 (((1,), (1,)), ((), ())),
                                 preferred_element_type=jnp.float32)


def _lm_head(x, emb):
    vb = 256
    return pl.pallas_call(
        _lm_head_kernel,
        grid=(V // vb,),
        in_specs=[pl.BlockSpec((T, D), lambda i: (0, 0)),
                  pl.BlockSpec((vb, D), lambda i: (i, 0))],
        out_specs=pl.BlockSpec((T, vb), lambda i: (0, i)),
        out_shape=jax.ShapeDtypeStruct((T, V), jnp.float32),
    )(x, emb)


def kernel(tokens, token_emb, pos_emb, Wr, Wq, Wk, Wv, Wo, g1, g2, W1, b1,
           W2, b2):
    idx = tokens.reshape(-1).astype(jnp.int32)
    emb = _sc_embed_gather(token_emb, idx)
    x = _add(emb, pos_emb[:T])
    for i in range(L):
        xn, sel3, csum3 = _norm_route(x, g1[i].reshape(1, D), Wr[i])
        q, k, v = _qkv(xn, Wq[i], Wk[i], Wv[i])
        q3 = q.reshape(T, H, DH).transpose(1, 0, 2)
        k3 = k.reshape(T, H, DH).transpose(1, 0, 2)
        v3 = v.reshape(T, H, DH).transpose(1, 0, 2)
        o3 = _head_attn(q3, k3, v3, sel3, csum3)
        attn = o3.transpose(1, 0, 2).reshape(T, D)
        x = _oproj_mlp(x, attn, Wo[i], g2[i].reshape(1, D), W1[i],
                       b1[i].reshape(1, 4 * D), W2[i], b2[i].reshape(1, D))
    logits = _lm_head(x, token_emb)
    return logits.reshape(1, T, V)
